# Initial kernel scaffold; baseline (speedup 1.0000x reference)
import jax
import jax.numpy as jnp
from jax.experimental import pallas as pl


def kernel(nv, ns, x, edge_index, Wms1, bms1, Wms2, bms2, Wmv, bmv, Wus1, bus1, Wus2, bus2):
    raise NotImplementedError("write your pallas kernel here")


# speedup vs baseline: 5.7713x; 5.7713x over previous
import math
import functools

import jax
import jax.numpy as jnp
from jax.experimental import pallas as pl

N = 10000
E = 160000
F = 128
L = 20
RC = 5.0
EPS = 1e-5
LOG2 = math.log(2.0)

NBLK = 1000
EBLK = 2000


def _ssp(x):
    return jax.nn.softplus(x) - LOG2


def _mlp1_body(ns_ref, w1_ref, b1_ref, w2_ref, b2_ref, phi_ref):
    h = jnp.dot(ns_ref[...], w1_ref[...], preferred_element_type=jnp.float32)
    h = _ssp(h + b1_ref[...])
    phi_ref[...] = (
        jnp.dot(h, w2_ref[...], preferred_element_type=jnp.float32) + b2_ref[...]
    )


def _mlp1(ns, W1, b1, W2, b2):
    return pl.pallas_call(
        _mlp1_body,
        grid=(N // NBLK,),
        in_specs=[
            pl.BlockSpec((NBLK, F), lambda i: (i, 0)),
            pl.BlockSpec((F, F), lambda i: (0, 0)),
            pl.BlockSpec((1, F), lambda i: (0, 0)),
            pl.BlockSpec((F, 3 * F), lambda i: (0, 0)),
            pl.BlockSpec((1, 3 * F), lambda i: (0, 0)),
        ],
        out_specs=pl.BlockSpec((NBLK, 3 * F), lambda i: (i, 0)),
        out_shape=jax.ShapeDtypeStruct((N, 3 * F), jnp.float32),
    )(ns, W1, b1[None, :], W2, b2[None, :])


def _edge_body(phie_ref, vjc_ref, xs_ref, xd_ref, wmv_ref, bmv_ref,
               dva_ref, ds_ref):
    xs = xs_ref[...]
    xd = xd_ref[...]
    vec = xs - xd
    r2 = jnp.sum(vec * vec, axis=-1, keepdims=True)
    r = jnp.sqrt(r2 + EPS)
    rnorm = jnp.sqrt(r * r + EPS)
    ls = 1.0 + jax.lax.broadcasted_iota(jnp.int32, (1, F), 1).astype(jnp.float32)
    rbf = jnp.sin((math.pi / RC) * (rnorm * ls)) / rnorm
    fc = 0.5 * (jnp.cos(math.pi * (r + EPS) / RC) + 1.0)
    w = fc * (jnp.dot(rbf, wmv_ref[...],
                      preferred_element_type=jnp.float32) + bmv_ref[...])
    msg = phie_ref[...] * w
    v_ = msg[:, 0:F]
    s_ = msg[:, F:2 * F]
    r_ = msg[:, 2 * F:3 * F]
    u = vec / r
    vjc = vjc_ref[...]
    dva_ref[:, 0:F] = vjc[:, 0:F] * v_ + r_ * u[:, 0:1]
    dva_ref[:, F:2 * F] = vjc[:, F:2 * F] * v_ + r_ * u[:, 1:2]
    dva_ref[:, 2 * F:3 * F] = vjc[:, 2 * F:3 * F] * v_ + r_ * u[:, 2:3]
    ds_ref[...] = s_


def _edge_stage(phiE, vjc, xs, xd, WmvP, bmv):
    return pl.pallas_call(
        _edge_body,
        grid=(E // EBLK,),
        in_specs=[
            pl.BlockSpec((EBLK, 3 * F), lambda i: (i, 0)),
            pl.BlockSpec((EBLK, 3 * F), lambda i: (i, 0)),
            pl.BlockSpec((EBLK, 3), lambda i: (i, 0)),
            pl.BlockSpec((EBLK, 3), lambda i: (i, 0)),
            pl.BlockSpec((F, 3 * F), lambda i: (0, 0)),
            pl.BlockSpec((1, 3 * F), lambda i: (0, 0)),
        ],
        out_specs=[
            pl.BlockSpec((EBLK, 3 * F), lambda i: (i, 0)),
            pl.BlockSpec((EBLK, F), lambda i: (i, 0)),
        ],
        out_shape=[
            jax.ShapeDtypeStruct((E, 3 * F), jnp.float32),
            jax.ShapeDtypeStruct((E, F), jnp.float32),
        ],
    )(phiE, vjc, xs, xd, WmvP, bmv[None, :])


def _mlp2_body(vnc_ref, sn_ref, w1_ref, b1_ref, w2_ref, b2_ref, s2_ref):
    vnc = vnc_ref[...]
    nrm = jnp.sqrt(vnc[:, 0:F] ** 2 + vnc[:, F:2 * F] ** 2
                   + vnc[:, 2 * F:3 * F] ** 2 + EPS)
    h = (jnp.dot(nrm, w1_ref[0:F, :], preferred_element_type=jnp.float32)
         + jnp.dot(sn_ref[...], w1_ref[F:2 * F, :],
                   preferred_element_type=jnp.float32))
    h = _ssp(h + b1_ref[...])
    s2_ref[...] = (
        jnp.dot(h, w2_ref[...], preferred_element_type=jnp.float32) + b2_ref[...]
    )


def _mlp2(vnc, s_new, W1, b1, W2, b2):
    return pl.pallas_call(
        _mlp2_body,
        grid=(N // NBLK,),
        in_specs=[
            pl.BlockSpec((NBLK, 3 * F), lambda i: (i, 0)),
            pl.BlockSpec((NBLK, F), lambda i: (i, 0)),
            pl.BlockSpec((2 * F, F), lambda i: (0, 0)),
            pl.BlockSpec((1, F), lambda i: (0, 0)),
            pl.BlockSpec((F, 3 * F), lambda i: (0, 0)),
            pl.BlockSpec((1, 3 * F), lambda i: (0, 0)),
        ],
        out_specs=pl.BlockSpec((NBLK, 3 * F), lambda i: (i, 0)),
        out_shape=jax.ShapeDtypeStruct((N, 3 * F), jnp.float32),
    )(vnc, s_new, W1, b1[None, :], W2, b2[None, :])


def _final_body(vnc_ref, sn_ref, uvs_ref, s2s_ref, deg_ref, vout_ref, sout_ref):
    invd = 1.0 / deg_ref[...]
    uvx = uvs_ref[:, 0:F] * invd
    uvy = uvs_ref[:, F:2 * F] * invd
    uvz = uvs_ref[:, 2 * F:3 * F] * invd
    smean_v = s2s_ref[:, 0:F] * invd
    smean_s = s2s_ref[:, F:2 * F] * invd
    smean_a = s2s_ref[:, 2 * F:3 * F] * invd
    s = uvx * uvx + uvy * uvy + uvz * uvz
    ds2 = s / (s + EPS) * smean_s + smean_a
    vnc = vnc_ref[...]
    vout_ref[:, 0:F] = vnc[:, 0:F] + uvx * smean_v
    vout_ref[:, F:2 * F] = vnc[:, F:2 * F] + uvy * smean_v
    vout_ref[:, 2 * F:3 * F] = vnc[:, 2 * F:3 * F] + uvz * smean_v
    sout_ref[...] = sn_ref[...] + ds2


def _final(vnc, s_new, uv_sum, s2_sum, deg):
    return pl.pallas_call(
        _final_body,
        grid=(N // NBLK,),
        in_specs=[
            pl.BlockSpec((NBLK, 3 * F), lambda i: (i, 0)),
            pl.BlockSpec((NBLK, F), lambda i: (i, 0)),
            pl.BlockSpec((NBLK, 3 * F), lambda i: (i, 0)),
            pl.BlockSpec((NBLK, 3 * F), lambda i: (i, 0)),
            pl.BlockSpec((NBLK, 1), lambda i: (i, 0)),
        ],
        out_specs=[
            pl.BlockSpec((NBLK, 3 * F), lambda i: (i, 0)),
            pl.BlockSpec((NBLK, F), lambda i: (i, 0)),
        ],
        out_shape=[
            jax.ShapeDtypeStruct((N, 3 * F), jnp.float32),
            jax.ShapeDtypeStruct((N, F), jnp.float32),
        ],
    )(vnc, s_new, uv_sum, s2_sum, deg)


def kernel(nv, ns, x, edge_index,
           Wms1, bms1, Wms2, bms2, Wmv, bmv, Wus1, bus1, Wus2, bus2):
    src = edge_index[0]
    dst = edge_index[1]

    phi = _mlp1(ns, Wms1, bms1, Wms2, bms2)

    WmvP = jnp.zeros((F, 3 * F), jnp.float32).at[0:L, :].set(Wmv)

    vnc0 = jnp.transpose(nv, (0, 2, 1)).reshape(N, 3 * F)

    phiE = jnp.take(phi, src, axis=0)
    vjc = jnp.take(vnc0, src, axis=0)
    xs = jnp.take(x, src, axis=0)
    xd = jnp.take(x, dst, axis=0)

    dva, ds_e = _edge_stage(phiE, vjc, xs, xd, WmvP, bmv)

    dv = jax.ops.segment_sum(dva, dst, num_segments=N)
    ds = jax.ops.segment_sum(ds_e, dst, num_segments=N)
    vnc = vnc0 + dv
    s_new = ns + ds

    s2 = _mlp2(vnc, s_new, Wus1, bus1, Wus2, bus2)

    ones = jnp.ones((E,), jnp.float32)
    deg = jnp.maximum(jax.ops.segment_sum(ones, dst, num_segments=N), 1.0)
    uv_sum = jax.ops.segment_sum(jnp.take(vnc, src, axis=0), dst, num_segments=N)
    s2_sum = jax.ops.segment_sum(jnp.take(s2, src, axis=0), dst, num_segments=N)

    vout_c, sout = _final(vnc, s_new, uv_sum, s2_sum, deg[:, None])
    vout = jnp.transpose(vout_c.reshape(N, 3, F), (0, 2, 1))
    return (vout, sout)
